# baseline (speedup 1.0000x reference)
import functools

import jax
import jax.numpy as jnp
from jax import lax
from jax.experimental import pallas as pl
from jax.experimental.pallas import tpu as pltpu
from jax.experimental.pallas import tpu_sc as plsc

N_NODES = 10000
N_EDGES = 160000
D = 256
DW = 16
NC, NS = 2, 16
R = 10112
RP = R // NS
EB = 128
NB = 40
CB = 8
NK = NB // CB
EP = NC * NS * NB * EB

_SC_PARAMS = pltpu.CompilerParams(use_tc_tiling_on_sc=False)


def _sc_aggregate(xb, src4, dst4, zacc):
    mesh = plsc.VectorSubcoreMesh(core_axis_name="c", subcore_axis_name="s")

    @functools.partial(
        pl.kernel,
        out_type=jax.ShapeDtypeStruct((NC, R, D), jnp.bfloat16),
        mesh=mesh,
        compiler_params=_SC_PARAMS,
        scratch_types=[
            pltpu.VMEM((CB, EB), jnp.int32),
            pltpu.VMEM((CB, EB), jnp.int32),
            pltpu.VMEM((EB, D), jnp.bfloat16),
            pltpu.VMEM((EB, D), jnp.bfloat16),
            pltpu.VMEM_SHARED((R, D), jnp.bfloat16),
            pltpu.SemaphoreType.DMA,
            pltpu.SemaphoreType.DMA,
        ],
    )
    def body(xb_hbm, src4_hbm, dst4_hbm, zacc_hbm, out_hbm, src_idx, dst_idx,
             rows0, rows1, acc_sh, sem0, sem1):
        c = lax.axis_index("c")
        s = lax.axis_index("s")
        pltpu.sync_copy(zacc_hbm, acc_sh.at[pl.ds(s * RP, RP)])
        plsc.subcore_barrier()

        def chunk(k, carry):
            pltpu.sync_copy(src4_hbm.at[c, s, pl.ds(k * CB, CB)], src_idx)
            pltpu.sync_copy(dst4_hbm.at[c, s, pl.ds(k * CB, CB)], dst_idx)
            pltpu.async_copy(xb_hbm.at[src_idx.at[0]], rows0, sem0)

            def pair(p, carry2):
                b0 = 2 * p

                pltpu.async_copy(xb_hbm.at[src_idx.at[b0 + 1]], rows1, sem1)
                pltpu.make_async_copy(xb_hbm.at[src_idx.at[b0]], rows0,
                                      sem0).wait()
                pltpu.sync_copy(rows0, acc_sh.at[dst_idx.at[b0]], add=True)

                @pl.when(p < CB // 2 - 1)
                def _():
                    pltpu.async_copy(xb_hbm.at[src_idx.at[b0 + 2]], rows0,
                                     sem0)

                pltpu.make_async_copy(xb_hbm.at[src_idx.at[b0 + 1]], rows1,
                                      sem1).wait()
                pltpu.sync_copy(rows1, acc_sh.at[dst_idx.at[b0 + 1]], add=True)
                return carry2

            lax.fori_loop(0, CB // 2, pair, 0)
            return carry

        lax.fori_loop(0, NK, chunk, 0)
        plsc.subcore_barrier()
        pltpu.sync_copy(acc_sh.at[pl.ds(s * RP, RP)],
                        out_hbm.at[c].at[pl.ds(s * RP, RP)])

    return body(xb, src4, dst4, zacc)


def _sc_degree(dst4, zdeg, ones16):
    mesh = plsc.VectorSubcoreMesh(core_axis_name="c", subcore_axis_name="s")

    @functools.partial(
        pl.kernel,
        out_type=jax.ShapeDtypeStruct((NC, R, DW), jnp.float32),
        mesh=mesh,
        compiler_params=_SC_PARAMS,
        scratch_types=[
            pltpu.VMEM((NB, EB), jnp.int32),
            pltpu.VMEM((EB, DW), jnp.float32),
            pltpu.VMEM_SHARED((R, DW), jnp.float32),
        ],
    )
    def body(dst4_hbm, zdeg_hbm, ones_hbm, deg_hbm, dst_idx, ones_v, deg_sh):
        c = lax.axis_index("c")
        s = lax.axis_index("s")
        pltpu.sync_copy(dst4_hbm.at[c].at[s], dst_idx)
        pltpu.sync_copy(ones_hbm, ones_v)
        pltpu.sync_copy(zdeg_hbm, deg_sh.at[pl.ds(s * RP, RP)])
        plsc.subcore_barrier()

        def batch(b, carry):
            pltpu.sync_copy(ones_v, deg_sh.at[dst_idx.at[b]], add=True)
            return carry

        lax.fori_loop(0, NB, batch, 0)
        plsc.subcore_barrier()
        pltpu.sync_copy(deg_sh.at[pl.ds(s * RP, RP)],
                        deg_hbm.at[c].at[pl.ds(s * RP, RP)])

    return body(dst4, zdeg, ones16)


def _tc_transport(agg, w, degp):
    MB = 1000

    def tc_body(a0_ref, a1_ref, w_ref, d0_ref, d1_ref, o_ref):
        af = (a0_ref[0].astype(jnp.float32) + a1_ref[0].astype(jnp.float32))
        y = lax.dot_general(af, w_ref[...], (((1,), (1,)), ((), ())),
                            preferred_element_type=jnp.float32)
        dg = d0_ref[0, :, 0:1] + d1_ref[0, :, 0:1]
        norm = jnp.where(dg > 0, lax.rsqrt(jnp.maximum(dg, 1.0)), 0.0)
        o_ref[...] = y * norm

    return pl.pallas_call(
        tc_body,
        grid=(N_NODES // MB,),
        in_specs=[
            pl.BlockSpec((1, MB, D), lambda i: (0, i, 0)),
            pl.BlockSpec((1, MB, D), lambda i: (1, i, 0)),
            pl.BlockSpec((D, D), lambda i: (0, 0)),
            pl.BlockSpec((1, MB, DW), lambda i: (0, i, 0)),
            pl.BlockSpec((1, MB, DW), lambda i: (1, i, 0)),
        ],
        out_specs=pl.BlockSpec((MB, D), lambda i: (i, 0)),
        out_shape=jax.ShapeDtypeStruct((N_NODES, D), jnp.float32),
    )(agg, agg, w, degp, degp)


def kernel(x, edge_index, W_connection):
    src = edge_index[0].astype(jnp.int32)
    dst = edge_index[1].astype(jnp.int32)
    pad = EP - N_EDGES
    srcp = jnp.concatenate([src, jnp.zeros((pad,), jnp.int32)])
    dstp = jnp.concatenate([dst, jnp.full((pad,), N_NODES, jnp.int32)])
    src4 = srcp.reshape(NC, NS, NB, EB)
    dst4 = dstp.reshape(NC, NS, NB, EB)

    xb = x.astype(jnp.bfloat16)
    zacc = jnp.zeros((RP, D), jnp.bfloat16)
    zdeg = jnp.zeros((RP, DW), jnp.float32)
    ones16 = jnp.ones((EB, DW), jnp.float32)

    agg = _sc_aggregate(xb, src4, dst4, zacc)
    degp = _sc_degree(dst4, zdeg, ones16)
    return _tc_transport(agg, W_connection, degp)


# speedup vs baseline: 2.0289x; 2.0289x over previous
import functools

import jax
import jax.numpy as jnp
from jax import lax
from jax.experimental import pallas as pl
from jax.experimental.pallas import tpu as pltpu
from jax.experimental.pallas import tpu_sc as plsc

N_NODES = 10000
N_EDGES = 160000
D = 256
DW = 16
NC, NS = 2, 16
R = 10112
RP = R // NS
EB = 128
NB = 40
CB = 8
NK = NB // CB
EP = NC * NS * NB * EB

_SC_PARAMS = pltpu.CompilerParams(use_tc_tiling_on_sc=False)


def _sc_aggregate(xb, src4, dst4, zacc):
    mesh = plsc.VectorSubcoreMesh(core_axis_name="c", subcore_axis_name="s")

    @functools.partial(
        pl.kernel,
        out_type=jax.ShapeDtypeStruct((NC, R, D), jnp.bfloat16),
        mesh=mesh,
        compiler_params=_SC_PARAMS,
        scratch_types=[
            pltpu.VMEM((CB, EB), jnp.int32),
            pltpu.VMEM((CB, EB), jnp.int32),
            pltpu.VMEM((EB, D), jnp.bfloat16),
            pltpu.VMEM((EB, D), jnp.bfloat16),
            pltpu.VMEM_SHARED((R, D), jnp.bfloat16),
            pltpu.SemaphoreType.DMA,
            pltpu.SemaphoreType.DMA,
        ],
    )
    def body(xb_hbm, src4_hbm, dst4_hbm, zacc_hbm, out_hbm, src_idx, dst_idx,
             rows0, rows1, acc_sh, sem0, sem1):
        c = lax.axis_index("c")
        s = lax.axis_index("s")
        pltpu.sync_copy(zacc_hbm, acc_sh.at[pl.ds(s * RP, RP)])
        plsc.subcore_barrier()

        def chunk(k, carry):
            pltpu.sync_copy(src4_hbm.at[c, s, pl.ds(k * CB, CB)], src_idx)
            pltpu.sync_copy(dst4_hbm.at[c, s, pl.ds(k * CB, CB)], dst_idx)
            pltpu.async_copy(xb_hbm.at[src_idx.at[0]], rows0, sem0)

            def pair(p, carry2):
                b0 = 2 * p

                pltpu.async_copy(xb_hbm.at[src_idx.at[b0 + 1]], rows1, sem1)
                pltpu.make_async_copy(xb_hbm.at[src_idx.at[b0]], rows0,
                                      sem0).wait()
                pltpu.sync_copy(rows0, acc_sh.at[dst_idx.at[b0]], add=True)

                @pl.when(p < CB // 2 - 1)
                def _():
                    pltpu.async_copy(xb_hbm.at[src_idx.at[b0 + 2]], rows0,
                                     sem0)

                pltpu.make_async_copy(xb_hbm.at[src_idx.at[b0 + 1]], rows1,
                                      sem1).wait()
                pltpu.sync_copy(rows1, acc_sh.at[dst_idx.at[b0 + 1]], add=True)
                return carry2

            lax.fori_loop(0, CB // 2, pair, 0)
            return carry

        lax.fori_loop(0, NK, chunk, 0)
        plsc.subcore_barrier()
        pltpu.sync_copy(acc_sh.at[pl.ds(s * RP, RP)],
                        out_hbm.at[c].at[pl.ds(s * RP, RP)])

    return body(xb, src4, dst4, zacc)


def _sc_degree(dst4, zdeg, ones16):
    mesh = plsc.VectorSubcoreMesh(core_axis_name="c", subcore_axis_name="s")

    @functools.partial(
        pl.kernel,
        out_type=jax.ShapeDtypeStruct((NC, R, DW), jnp.float32),
        mesh=mesh,
        compiler_params=_SC_PARAMS,
        scratch_types=[
            pltpu.VMEM((NB, EB), jnp.int32),
            pltpu.VMEM((EB, DW), jnp.float32),
            pltpu.VMEM_SHARED((R, DW), jnp.float32),
        ],
    )
    def body(dst4_hbm, zdeg_hbm, ones_hbm, deg_hbm, dst_idx, ones_v, deg_sh):
        c = lax.axis_index("c")
        s = lax.axis_index("s")
        pltpu.sync_copy(dst4_hbm.at[c].at[s], dst_idx)
        pltpu.sync_copy(ones_hbm, ones_v)
        pltpu.sync_copy(zdeg_hbm, deg_sh.at[pl.ds(s * RP, RP)])
        plsc.subcore_barrier()

        def batch(b, carry):
            pltpu.sync_copy(ones_v, deg_sh.at[dst_idx.at[b]], add=True)
            return carry

        lax.fori_loop(0, NB, batch, 0)
        plsc.subcore_barrier()
        pltpu.sync_copy(deg_sh.at[pl.ds(s * RP, RP)],
                        deg_hbm.at[c].at[pl.ds(s * RP, RP)])

    return body(dst4, zdeg, ones16)


def _tc_transport(agg, w, degp):
    MB = 1000

    def tc_body(a0_ref, a1_ref, w_ref, d0_ref, d1_ref, o_ref):
        af = (a0_ref[0].astype(jnp.float32) + a1_ref[0].astype(jnp.float32))
        y = lax.dot_general(af, w_ref[...], (((1,), (1,)), ((), ())),
                            preferred_element_type=jnp.float32)
        dg = d0_ref[0, :, 0:1] + d1_ref[0, :, 0:1]
        norm = jnp.where(dg > 0, lax.rsqrt(jnp.maximum(dg, 1.0)), 0.0)
        o_ref[...] = y * norm

    return pl.pallas_call(
        tc_body,
        grid=(N_NODES // MB,),
        in_specs=[
            pl.BlockSpec((1, MB, D), lambda i: (0, i, 0)),
            pl.BlockSpec((1, MB, D), lambda i: (1, i, 0)),
            pl.BlockSpec((D, D), lambda i: (0, 0)),
            pl.BlockSpec((1, MB, DW), lambda i: (0, i, 0)),
            pl.BlockSpec((1, MB, DW), lambda i: (1, i, 0)),
        ],
        out_specs=pl.BlockSpec((MB, D), lambda i: (i, 0)),
        out_shape=jax.ShapeDtypeStruct((N_NODES, D), jnp.float32),
    )(agg, agg, w, degp, degp)


def kernel(x, edge_index, W_connection):
    src = edge_index[0].astype(jnp.int32)
    dst = edge_index[1].astype(jnp.int32)
    pad = EP - N_EDGES
    ar = jnp.arange(pad, dtype=jnp.int32)
    srcp = jnp.concatenate([src, ar % N_NODES])
    dstp = jnp.concatenate([dst, N_NODES + ar % (R - N_NODES)])
    src4 = srcp.reshape(NC, NS, NB, EB)
    dst4 = dstp.reshape(NC, NS, NB, EB)

    xb = x.astype(jnp.bfloat16)
    zacc = jnp.zeros((RP, D), jnp.bfloat16)
    zdeg = jnp.zeros((RP, DW), jnp.float32)
    ones16 = jnp.ones((EB, DW), jnp.float32)

    agg = _sc_aggregate(xb, src4, dst4, zacc)
    degp = _sc_degree(dst4, zdeg, ones16)
    return _tc_transport(agg, W_connection, degp)


# speedup vs baseline: 2.0892x; 1.0297x over previous
import functools

import jax
import jax.numpy as jnp
from jax import lax
from jax.experimental import pallas as pl
from jax.experimental.pallas import tpu as pltpu
from jax.experimental.pallas import tpu_sc as plsc

N_NODES = 10000
N_EDGES = 160000
D = 256
DW = 16
NC, NS = 2, 16
R = 10112
RP = R // NS
EB = 128
NB = 40
CB = 20
NK = NB // CB
EP = NC * NS * NB * EB

_SC_PARAMS = pltpu.CompilerParams(use_tc_tiling_on_sc=False)


def _sc_aggregate(xb, src4, dst4, zacc, zdeg, ones16):
    mesh = plsc.VectorSubcoreMesh(core_axis_name="c", subcore_axis_name="s")

    @functools.partial(
        pl.kernel,
        out_type=(jax.ShapeDtypeStruct((NC, R, D), jnp.bfloat16),
                  jax.ShapeDtypeStruct((NC, R, DW), jnp.float32)),
        mesh=mesh,
        compiler_params=_SC_PARAMS,
        scratch_types=[
            pltpu.VMEM((CB, EB), jnp.int32),
            pltpu.VMEM((CB, EB), jnp.int32),
            pltpu.VMEM((EB, D), jnp.bfloat16),
            pltpu.VMEM((EB, D), jnp.bfloat16),
            pltpu.VMEM((EB, DW), jnp.float32),
            pltpu.VMEM_SHARED((R, D), jnp.bfloat16),
            pltpu.VMEM_SHARED((R, DW), jnp.float32),
            pltpu.SemaphoreType.DMA,
            pltpu.SemaphoreType.DMA,
        ],
    )
    def body(xb_hbm, src4_hbm, dst4_hbm, zacc_hbm, zdeg_hbm, ones_hbm,
             out_hbm, deg_hbm, src_idx, dst_idx, rows0, rows1, ones_v,
             acc_sh, deg_sh, sem0, sem1):
        c = lax.axis_index("c")
        s = lax.axis_index("s")
        pltpu.sync_copy(zacc_hbm, acc_sh.at[pl.ds(s * RP, RP)])
        pltpu.sync_copy(zdeg_hbm, deg_sh.at[pl.ds(s * RP, RP)])
        pltpu.sync_copy(ones_hbm, ones_v)
        plsc.subcore_barrier()

        def chunk(k, carry):
            pltpu.sync_copy(src4_hbm.at[c, s, pl.ds(k * CB, CB)], src_idx)
            pltpu.sync_copy(dst4_hbm.at[c, s, pl.ds(k * CB, CB)], dst_idx)
            pltpu.async_copy(xb_hbm.at[src_idx.at[0]], rows0, sem0)

            def pair(p, carry2):
                b0 = 2 * p

                pltpu.async_copy(xb_hbm.at[src_idx.at[b0 + 1]], rows1, sem1)
                pltpu.make_async_copy(xb_hbm.at[src_idx.at[b0]], rows0,
                                      sem0).wait()
                pltpu.sync_copy(rows0, acc_sh.at[dst_idx.at[b0]], add=True)
                pltpu.sync_copy(ones_v, deg_sh.at[dst_idx.at[b0]], add=True)

                @pl.when(p < CB // 2 - 1)
                def _():
                    pltpu.async_copy(xb_hbm.at[src_idx.at[b0 + 2]], rows0,
                                     sem0)

                pltpu.make_async_copy(xb_hbm.at[src_idx.at[b0 + 1]], rows1,
                                      sem1).wait()
                pltpu.sync_copy(rows1, acc_sh.at[dst_idx.at[b0 + 1]], add=True)
                pltpu.sync_copy(ones_v, deg_sh.at[dst_idx.at[b0 + 1]],
                                add=True)
                return carry2

            lax.fori_loop(0, CB // 2, pair, 0)
            return carry

        lax.fori_loop(0, NK, chunk, 0)
        plsc.subcore_barrier()
        pltpu.sync_copy(acc_sh.at[pl.ds(s * RP, RP)],
                        out_hbm.at[c].at[pl.ds(s * RP, RP)])
        pltpu.sync_copy(deg_sh.at[pl.ds(s * RP, RP)],
                        deg_hbm.at[c].at[pl.ds(s * RP, RP)])

    return body(xb, src4, dst4, zacc, zdeg, ones16)


def _tc_transport(agg, w, degp):
    MB = 1000

    def tc_body(a0_ref, a1_ref, w_ref, d0_ref, d1_ref, o_ref):
        af = (a0_ref[0].astype(jnp.float32) + a1_ref[0].astype(jnp.float32))
        y = lax.dot_general(af, w_ref[...], (((1,), (1,)), ((), ())),
                            preferred_element_type=jnp.float32)
        dg = d0_ref[0, :, 0:1] + d1_ref[0, :, 0:1]
        norm = jnp.where(dg > 0, lax.rsqrt(jnp.maximum(dg, 1.0)), 0.0)
        o_ref[...] = y * norm

    return pl.pallas_call(
        tc_body,
        grid=(N_NODES // MB,),
        in_specs=[
            pl.BlockSpec((1, MB, D), lambda i: (0, i, 0)),
            pl.BlockSpec((1, MB, D), lambda i: (1, i, 0)),
            pl.BlockSpec((D, D), lambda i: (0, 0)),
            pl.BlockSpec((1, MB, DW), lambda i: (0, i, 0)),
            pl.BlockSpec((1, MB, DW), lambda i: (1, i, 0)),
        ],
        out_specs=pl.BlockSpec((MB, D), lambda i: (i, 0)),
        out_shape=jax.ShapeDtypeStruct((N_NODES, D), jnp.float32),
    )(agg, agg, w, degp, degp)


def kernel(x, edge_index, W_connection):
    src = edge_index[0].astype(jnp.int32)
    dst = edge_index[1].astype(jnp.int32)
    pad = EP - N_EDGES
    ar = jnp.arange(pad, dtype=jnp.int32)
    srcp = jnp.concatenate([src, ar % N_NODES])
    dstp = jnp.concatenate([dst, N_NODES + ar % (R - N_NODES)])
    src4 = srcp.reshape(NC, NS, NB, EB)
    dst4 = dstp.reshape(NC, NS, NB, EB)

    xb = x.astype(jnp.bfloat16)
    zacc = jnp.zeros((RP, D), jnp.bfloat16)
    zdeg = jnp.zeros((RP, DW), jnp.float32)
    ones16 = jnp.ones((EB, DW), jnp.float32)

    agg, degp = _sc_aggregate(xb, src4, dst4, zacc, zdeg, ones16)
    return _tc_transport(agg, W_connection, degp)


# speedup vs baseline: 2.0895x; 1.0001x over previous
import functools

import jax
import jax.numpy as jnp
from jax import lax
from jax.experimental import pallas as pl
from jax.experimental.pallas import tpu as pltpu
from jax.experimental.pallas import tpu_sc as plsc

N_NODES = 10000
N_EDGES = 160000
D = 256
DW = 16
NC, NS = 2, 16
R = 10112
RP = R // NS
EB = 128
NB = 40
CB = 20
NK = NB // CB
EP = NC * NS * NB * EB

_SC_PARAMS = pltpu.CompilerParams(use_tc_tiling_on_sc=False)


def _sc_aggregate(xb, src4, dst4, zacc, zdeg, ones16):
    mesh = plsc.VectorSubcoreMesh(core_axis_name="c", subcore_axis_name="s")

    @functools.partial(
        pl.kernel,
        out_type=(jax.ShapeDtypeStruct((NC, R, D), jnp.bfloat16),
                  jax.ShapeDtypeStruct((NC, R, DW), jnp.float32)),
        mesh=mesh,
        compiler_params=_SC_PARAMS,
        scratch_types=[
            pltpu.VMEM((CB, EB), jnp.int32),
            pltpu.VMEM((CB, EB), jnp.int32),
            pltpu.VMEM((EB, D), jnp.bfloat16),
            pltpu.VMEM((EB, D), jnp.bfloat16),
            pltpu.VMEM((EB, DW), jnp.float32),
            pltpu.VMEM_SHARED((R, D), jnp.bfloat16),
            pltpu.VMEM_SHARED((R, DW), jnp.float32),
            pltpu.SemaphoreType.DMA,
            pltpu.SemaphoreType.DMA,
        ],
    )
    def body(xb_hbm, src4_hbm, dst4_hbm, zacc_hbm, zdeg_hbm, ones_hbm,
             out_hbm, deg_hbm, src_idx, dst_idx, rows0, rows1, ones_v,
             acc_sh, deg_sh, sem0, sem1):
        c = lax.axis_index("c")
        s = lax.axis_index("s")
        pltpu.sync_copy(zacc_hbm, acc_sh.at[pl.ds(s * RP, RP)])
        pltpu.sync_copy(zdeg_hbm, deg_sh.at[pl.ds(s * RP, RP)])
        pltpu.sync_copy(ones_hbm, ones_v)
        plsc.subcore_barrier()

        def chunk(k, carry):
            pltpu.sync_copy(src4_hbm.at[c, s, pl.ds(k * CB, CB)], src_idx)
            pltpu.sync_copy(dst4_hbm.at[c, s, pl.ds(k * CB, CB)], dst_idx)
            pltpu.async_copy(xb_hbm.at[src_idx.at[0]], rows0, sem0)

            def pair(p, carry2):
                b0 = 2 * p

                pltpu.async_copy(xb_hbm.at[src_idx.at[b0 + 1]], rows1, sem1)
                pltpu.make_async_copy(xb_hbm.at[src_idx.at[b0]], rows0,
                                      sem0).wait()
                pltpu.sync_copy(rows0, acc_sh.at[dst_idx.at[b0]], add=True)
                pltpu.sync_copy(ones_v, deg_sh.at[dst_idx.at[b0]], add=True)

                @pl.when(p < CB // 2 - 1)
                def _():
                    pltpu.async_copy(xb_hbm.at[src_idx.at[b0 + 2]], rows0,
                                     sem0)

                pltpu.make_async_copy(xb_hbm.at[src_idx.at[b0 + 1]], rows1,
                                      sem1).wait()
                pltpu.sync_copy(rows1, acc_sh.at[dst_idx.at[b0 + 1]], add=True)
                pltpu.sync_copy(ones_v, deg_sh.at[dst_idx.at[b0 + 1]],
                                add=True)
                return carry2

            lax.fori_loop(0, CB // 2, pair, 0)
            return carry

        lax.fori_loop(0, NK, chunk, 0)
        plsc.subcore_barrier()
        pltpu.sync_copy(acc_sh.at[pl.ds(s * RP, RP)],
                        out_hbm.at[c].at[pl.ds(s * RP, RP)])
        pltpu.sync_copy(deg_sh.at[pl.ds(s * RP, RP)],
                        deg_hbm.at[c].at[pl.ds(s * RP, RP)])

    return body(xb, src4, dst4, zacc, zdeg, ones16)


def _tc_transport(agg, w, degp):
    MB = 1000

    def tc_body(a_ref, w_ref, d_ref, o_ref):
        af = (a_ref[0].astype(jnp.float32) + a_ref[1].astype(jnp.float32))
        y = lax.dot_general(af, w_ref[...], (((1,), (1,)), ((), ())),
                            preferred_element_type=jnp.float32)
        dg = d_ref[0, :, 0:1] + d_ref[1, :, 0:1]
        norm = jnp.where(dg > 0, lax.rsqrt(jnp.maximum(dg, 1.0)), 0.0)
        o_ref[...] = y * norm

    return pl.pallas_call(
        tc_body,
        grid=(N_NODES // MB,),
        in_specs=[
            pl.BlockSpec((NC, MB, D), lambda i: (0, i, 0)),
            pl.BlockSpec((D, D), lambda i: (0, 0)),
            pl.BlockSpec((NC, MB, DW), lambda i: (0, i, 0)),
        ],
        out_specs=pl.BlockSpec((MB, D), lambda i: (i, 0)),
        out_shape=jax.ShapeDtypeStruct((N_NODES, D), jnp.float32),
    )(agg, w, degp)


def kernel(x, edge_index, W_connection):
    src = edge_index[0].astype(jnp.int32)
    dst = edge_index[1].astype(jnp.int32)
    pad = EP - N_EDGES
    ar = jnp.arange(pad, dtype=jnp.int32)
    srcp = jnp.concatenate([src, ar % N_NODES])
    dstp = jnp.concatenate([dst, N_NODES + ar % (R - N_NODES)])
    src4 = srcp.reshape(NC, NS, NB, EB)
    dst4 = dstp.reshape(NC, NS, NB, EB)

    xb = x.astype(jnp.bfloat16)
    zacc = jnp.zeros((RP, D), jnp.bfloat16)
    zdeg = jnp.zeros((RP, DW), jnp.float32)
    ones16 = jnp.ones((EB, DW), jnp.float32)

    agg, degp = _sc_aggregate(xb, src4, dst4, zacc, zdeg, ones16)
    return _tc_transport(agg, W_connection, degp)


# speedup vs baseline: 2.1076x; 1.0087x over previous
import functools

import jax
import jax.numpy as jnp
from jax import lax
from jax.experimental import pallas as pl
from jax.experimental.pallas import tpu as pltpu
from jax.experimental.pallas import tpu_sc as plsc

N_NODES = 10000
N_EDGES = 160000
D = 256
DW = 16
NC, NS = 2, 16
R = 10112
RP = R // NS
EB = 128
NB = 40
CB = 20
NK = NB // CB
EP = NC * NS * NB * EB

_SC_PARAMS = pltpu.CompilerParams(use_tc_tiling_on_sc=False)


def _sc_aggregate(xb, src4, dst4, zacc, zdeg, ones16):
    mesh = plsc.VectorSubcoreMesh(core_axis_name="c", subcore_axis_name="s")

    @functools.partial(
        pl.kernel,
        out_type=(jax.ShapeDtypeStruct((NC, R, D), jnp.bfloat16),
                  jax.ShapeDtypeStruct((NC, R, DW), jnp.float32)),
        mesh=mesh,
        compiler_params=_SC_PARAMS,
        scratch_types=[
            pltpu.VMEM((CB, EB), jnp.int32),
            pltpu.VMEM((CB, EB), jnp.int32),
            pltpu.VMEM((EB, D), jnp.bfloat16),
            pltpu.VMEM((EB, D), jnp.bfloat16),
            pltpu.VMEM((EB, DW), jnp.float32),
            pltpu.VMEM_SHARED((R, D), jnp.bfloat16),
            pltpu.VMEM_SHARED((R, DW), jnp.float32),
            pltpu.SemaphoreType.DMA,
            pltpu.SemaphoreType.DMA,
        ],
    )
    def body(xb_hbm, src4_hbm, dst4_hbm, zacc_hbm, zdeg_hbm, ones_hbm,
             out_hbm, deg_hbm, src_idx, dst_idx, rows0, rows1, ones_v,
             acc_sh, deg_sh, sem0, sem1):
        c = lax.axis_index("c")
        s = lax.axis_index("s")
        pltpu.sync_copy(src4_hbm.at[c, s, pl.ds(0, CB)], src_idx)
        pltpu.sync_copy(dst4_hbm.at[c, s, pl.ds(0, CB)], dst_idx)
        pltpu.async_copy(xb_hbm.at[src_idx.at[0]], rows0, sem0)
        pltpu.async_copy(xb_hbm.at[src_idx.at[1]], rows1, sem1)
        pltpu.sync_copy(zacc_hbm, acc_sh.at[pl.ds(s * RP, RP)])
        pltpu.sync_copy(zdeg_hbm, deg_sh.at[pl.ds(s * RP, RP)])
        pltpu.sync_copy(ones_hbm, ones_v)
        plsc.subcore_barrier()

        def chunk(k, carry):
            @pl.when(k > 0)
            def _():
                pltpu.sync_copy(src4_hbm.at[c, s, pl.ds(k * CB, CB)], src_idx)
                pltpu.sync_copy(dst4_hbm.at[c, s, pl.ds(k * CB, CB)], dst_idx)
                pltpu.async_copy(xb_hbm.at[src_idx.at[0]], rows0, sem0)
                pltpu.async_copy(xb_hbm.at[src_idx.at[1]], rows1, sem1)

            def pair(p, carry2):
                b0 = 2 * p
                last = p >= CB // 2 - 1

                pltpu.make_async_copy(xb_hbm.at[src_idx.at[b0]], rows0,
                                      sem0).wait()
                pltpu.sync_copy(rows0, acc_sh.at[dst_idx.at[b0]], add=True)
                pltpu.sync_copy(ones_v, deg_sh.at[dst_idx.at[b0]], add=True)

                @pl.when(jnp.logical_not(last))
                def _():
                    pltpu.async_copy(xb_hbm.at[src_idx.at[b0 + 2]], rows0,
                                     sem0)

                pltpu.make_async_copy(xb_hbm.at[src_idx.at[b0 + 1]], rows1,
                                      sem1).wait()
                pltpu.sync_copy(rows1, acc_sh.at[dst_idx.at[b0 + 1]], add=True)
                pltpu.sync_copy(ones_v, deg_sh.at[dst_idx.at[b0 + 1]],
                                add=True)

                @pl.when(jnp.logical_not(last))
                def _():
                    pltpu.async_copy(xb_hbm.at[src_idx.at[b0 + 3]], rows1,
                                     sem1)

                return carry2

            lax.fori_loop(0, CB // 2, pair, 0)
            return carry

        lax.fori_loop(0, NK, chunk, 0)
        plsc.subcore_barrier()
        pltpu.sync_copy(acc_sh.at[pl.ds(s * RP, RP)],
                        out_hbm.at[c].at[pl.ds(s * RP, RP)])
        pltpu.sync_copy(deg_sh.at[pl.ds(s * RP, RP)],
                        deg_hbm.at[c].at[pl.ds(s * RP, RP)])

    return body(xb, src4, dst4, zacc, zdeg, ones16)


def _tc_transport(agg, w, degp):
    MB = 1000

    def tc_body(a_ref, w_ref, d_ref, o_ref):
        af = (a_ref[0].astype(jnp.float32) + a_ref[1].astype(jnp.float32))
        y = lax.dot_general(af, w_ref[...], (((1,), (1,)), ((), ())),
                            preferred_element_type=jnp.float32)
        dg = d_ref[0, :, 0:1] + d_ref[1, :, 0:1]
        norm = jnp.where(dg > 0, lax.rsqrt(jnp.maximum(dg, 1.0)), 0.0)
        o_ref[...] = y * norm

    return pl.pallas_call(
        tc_body,
        grid=(N_NODES // MB,),
        in_specs=[
            pl.BlockSpec((NC, MB, D), lambda i: (0, i, 0)),
            pl.BlockSpec((D, D), lambda i: (0, 0)),
            pl.BlockSpec((NC, MB, DW), lambda i: (0, i, 0)),
        ],
        out_specs=pl.BlockSpec((MB, D), lambda i: (i, 0)),
        out_shape=jax.ShapeDtypeStruct((N_NODES, D), jnp.float32),
    )(agg, w, degp)


def kernel(x, edge_index, W_connection):
    src = edge_index[0].astype(jnp.int32)
    dst = edge_index[1].astype(jnp.int32)
    pad = EP - N_EDGES
    ar = jnp.arange(pad, dtype=jnp.int32)
    srcp = jnp.concatenate([src, ar % N_NODES])
    dstp = jnp.concatenate([dst, N_NODES + ar % (R - N_NODES)])
    src4 = srcp.reshape(NC, NS, NB, EB)
    dst4 = dstp.reshape(NC, NS, NB, EB)

    xb = x.astype(jnp.bfloat16)
    zacc = jnp.zeros((RP, D), jnp.bfloat16)
    zdeg = jnp.zeros((RP, DW), jnp.float32)
    ones16 = jnp.ones((EB, DW), jnp.float32)

    agg, degp = _sc_aggregate(xb, src4, dst4, zacc, zdeg, ones16)
    return _tc_transport(agg, W_connection, degp)
